# initial kernel scaffold (unmeasured)
import jax
import jax.numpy as jnp
from jax import lax
from jax.experimental import pallas as pl
from jax.experimental.pallas import tpu as pltpu

B, SQ, H, D = 4, 32, 8, 128
SKV_SHARD = 4096
KL = SKV_SHARD // 2
SCALE = D ** -0.5


def kernel(Q, K, V):
    def body(q_ref, k_ref, v_ref, out_ref,
             k_bufs, v_bufs, acc_m, acc_l, acc_o,
             rx_m, rx_l, rx_o,
             kv_sems, send_sems, recv_sems):
        my_x = lax.axis_index("x")
        my_y = lax.axis_index("y")
        x_partner = (1 - my_x, my_y)
        y_partner = (my_x, 1 - my_y)

        barrier = pltpu.get_barrier_semaphore()
        for dev in (x_partner, y_partner):
            pl.semaphore_signal(barrier, inc=1, device_id=dev,
                                device_id_type=pl.DeviceIdType.MESH)
        pl.semaphore_wait(barrier, 2)

        xoff = my_x * KL

        for b in range(B):
            copies = []
            for h in range(H):
                ck = pltpu.make_async_copy(
                    k_ref.at[b, pl.ds(xoff, KL), h, :], k_bufs.at[h],
                    kv_sems.at[0, h])
                cv = pltpu.make_async_copy(
                    v_ref.at[b, pl.ds(xoff, KL), h, :], v_bufs.at[h],
                    kv_sems.at[1, h])
                ck.start()
                cv.start()
                copies.append((ck, cv))
            qb = q_ref[b]
            for h in range(H):
                ck, cv = copies[h]
                ck.wait()
                cv.wait()
                qbh = qb[:, h, :]
                s = lax.dot_general(
                    qbh, k_bufs[h], (((1,), (1,)), ((), ())),
                    preferred_element_type=jnp.float32) * SCALE
                m = jnp.max(s, axis=1)
                p = jnp.exp(s - m[:, None])
                l = jnp.sum(p, axis=1)
                o = lax.dot_general(
                    p, v_bufs[h], (((1,), (0,)), ((), ())),
                    preferred_element_type=jnp.float32)
                acc_m[b, h, :] = m
                acc_l[b, h, :] = l
                acc_o[b, h, :, :] = o

        for stage, partner in enumerate((x_partner, y_partner)):
            rd_m = pltpu.make_async_remote_copy(
                src_ref=acc_m, dst_ref=rx_m.at[stage],
                send_sem=send_sems.at[stage, 0],
                recv_sem=recv_sems.at[stage, 0],
                device_id=partner, device_id_type=pl.DeviceIdType.MESH)
            rd_l = pltpu.make_async_remote_copy(
                src_ref=acc_l, dst_ref=rx_l.at[stage],
                send_sem=send_sems.at[stage, 1],
                recv_sem=recv_sems.at[stage, 1],
                device_id=partner, device_id_type=pl.DeviceIdType.MESH)
            rd_o = pltpu.make_async_remote_copy(
                src_ref=acc_o, dst_ref=rx_o.at[stage],
                send_sem=send_sems.at[stage, 2],
                recv_sem=recv_sems.at[stage, 2],
                device_id=partner, device_id_type=pl.DeviceIdType.MESH)
            rd_m.start()
            rd_l.start()
            rd_o.start()
            rd_m.wait()
            rd_l.wait()
            rd_o.wait()

            m1 = acc_m[...]
            m2 = rx_m[stage]
            mn = jnp.maximum(m1, m2)
            a1 = jnp.exp(m1 - mn)
            a2 = jnp.exp(m2 - mn)
            acc_m[...] = mn
            acc_l[...] = acc_l[...] * a1 + rx_l[stage] * a2
            acc_o[...] = (acc_o[...] * a1[..., None]
                          + rx_o[stage] * a2[..., None])

        for b in range(B):
            for h in range(H):
                out_ref[b, :, h, :] = acc_o[b, h] / acc_l[b, h][:, None]

    return pl.pallas_call(
        body,
        out_shape=jax.ShapeDtypeStruct((B, SQ, H, D), jnp.float32),
        in_specs=[
            pl.BlockSpec(memory_space=pltpu.VMEM),
            pl.BlockSpec(memory_space=pltpu.ANY),
            pl.BlockSpec(memory_space=pltpu.ANY),
        ],
        out_specs=pl.BlockSpec(memory_space=pltpu.VMEM),
        scratch_shapes=[
            pltpu.VMEM((H, KL, D), jnp.float32),
            pltpu.VMEM((H, KL, D), jnp.float32),
            pltpu.VMEM((B, H, SQ), jnp.float32),
            pltpu.VMEM((B, H, SQ), jnp.float32),
            pltpu.VMEM((B, H, SQ, D), jnp.float32),
            pltpu.VMEM((2, B, H, SQ), jnp.float32),
            pltpu.VMEM((2, B, H, SQ), jnp.float32),
            pltpu.VMEM((2, B, H, SQ, D), jnp.float32),
            pltpu.SemaphoreType.DMA((2, H)),
            pltpu.SemaphoreType.DMA((2, 3)),
            pltpu.SemaphoreType.DMA((2, 3)),
        ],
        compiler_params=pltpu.CompilerParams(collective_id=0),
    )(Q, K, V)


# baseline (device time: 47572 ns/iter reference)
import jax
import jax.numpy as jnp
from jax import lax
from jax.experimental import pallas as pl
from jax.experimental.pallas import tpu as pltpu

B, SQ, H, D = 4, 32, 8, 128
SKV_SHARD = 4096
KL = SKV_SHARD // 2
SCALE = D ** -0.5


def kernel(Q, K, V):
    def body(q_ref, k_ref, v_ref, out_ref,
             k_bufs, v_bufs, acc_m, acc_l, acc_o,
             rx_m, rx_l, rx_o,
             kv_sems, send_sems, recv_sems):
        my_x = lax.axis_index("x")
        my_y = lax.axis_index("y")
        x_partner = (1 - my_x, my_y)
        y_partner = (my_x, 1 - my_y)

        barrier = pltpu.get_barrier_semaphore()
        for dev in (x_partner, y_partner):
            pl.semaphore_signal(barrier, inc=1, device_id=dev,
                                device_id_type=pl.DeviceIdType.MESH)
        pl.semaphore_wait(barrier, 2)

        xoff = my_x * KL

        for b in range(B):
            copies = []
            for h in range(H):
                ck = pltpu.make_async_copy(
                    k_ref.at[b, pl.ds(xoff, KL), h, :], k_bufs.at[h],
                    kv_sems.at[0, h])
                cv = pltpu.make_async_copy(
                    v_ref.at[b, pl.ds(xoff, KL), h, :], v_bufs.at[h],
                    kv_sems.at[1, h])
                ck.start()
                cv.start()
                copies.append((ck, cv))
            qb = q_ref[b]
            for h in range(H):
                ck, cv = copies[h]
                ck.wait()
                cv.wait()
                qbh = qb[:, h, :]
                s = lax.dot_general(
                    qbh, k_bufs[h], (((1,), (1,)), ((), ())),
                    preferred_element_type=jnp.float32) * SCALE
                m = jnp.max(s, axis=1)
                p = jnp.exp(s - m[:, None])
                l = jnp.sum(p, axis=1)
                o = lax.dot_general(
                    p, v_bufs[h], (((1,), (0,)), ((), ())),
                    preferred_element_type=jnp.float32)
                acc_m[b, h, :] = m
                acc_l[b, h, :] = l
                acc_o[b, h, :, :] = o

        for stage, partner in enumerate((x_partner, y_partner)):
            rd_m = pltpu.make_async_remote_copy(
                src_ref=acc_m, dst_ref=rx_m.at[stage],
                send_sem=send_sems.at[stage, 0],
                recv_sem=recv_sems.at[stage, 0],
                device_id=partner, device_id_type=pl.DeviceIdType.MESH)
            rd_l = pltpu.make_async_remote_copy(
                src_ref=acc_l, dst_ref=rx_l.at[stage],
                send_sem=send_sems.at[stage, 1],
                recv_sem=recv_sems.at[stage, 1],
                device_id=partner, device_id_type=pl.DeviceIdType.MESH)
            rd_o = pltpu.make_async_remote_copy(
                src_ref=acc_o, dst_ref=rx_o.at[stage],
                send_sem=send_sems.at[stage, 2],
                recv_sem=recv_sems.at[stage, 2],
                device_id=partner, device_id_type=pl.DeviceIdType.MESH)
            rd_m.start()
            rd_l.start()
            rd_o.start()
            rd_m.wait()
            rd_l.wait()
            rd_o.wait()

            m1 = acc_m[...]
            m2 = rx_m[stage]
            mn = jnp.maximum(m1, m2)
            a1 = jnp.exp(m1 - mn)
            a2 = jnp.exp(m2 - mn)
            acc_m[...] = mn
            acc_l[...] = acc_l[...] * a1 + rx_l[stage] * a2
            acc_o[...] = (acc_o[...] * a1[..., None]
                          + rx_o[stage] * a2[..., None])

        for b in range(B):
            for h in range(H):
                out_ref[b, :, h, :] = acc_o[b, h] / acc_l[b, h][:, None]

    return pl.pallas_call(
        body,
        out_shape=jax.ShapeDtypeStruct((B, SQ, H, D), jnp.float32),
        in_specs=[
            pl.BlockSpec(memory_space=pltpu.VMEM),
            pl.BlockSpec(memory_space=pl.ANY),
            pl.BlockSpec(memory_space=pl.ANY),
        ],
        out_specs=pl.BlockSpec(memory_space=pltpu.VMEM),
        scratch_shapes=[
            pltpu.VMEM((H, KL, D), jnp.float32),
            pltpu.VMEM((H, KL, D), jnp.float32),
            pltpu.VMEM((B, H, SQ), jnp.float32),
            pltpu.VMEM((B, H, SQ), jnp.float32),
            pltpu.VMEM((B, H, SQ, D), jnp.float32),
            pltpu.VMEM((2, B, H, SQ), jnp.float32),
            pltpu.VMEM((2, B, H, SQ), jnp.float32),
            pltpu.VMEM((2, B, H, SQ, D), jnp.float32),
            pltpu.SemaphoreType.DMA((2, H)),
            pltpu.SemaphoreType.DMA((2, 3)),
            pltpu.SemaphoreType.DMA((2, 3)),
        ],
        compiler_params=pltpu.CompilerParams(collective_id=0),
    )(Q, K, V)
